# baseline (speedup 1.0000x reference)
import functools

import jax
import jax.numpy as jnp
from jax import lax
from jax.experimental import pallas as pl
from jax.experimental.pallas import tpu as pltpu
from jax.experimental.pallas import tpu_sc as plsc

N = 10000
D = 128
FH = 64
E = 320000
NC = 2
NS = 16
NW = NC * NS
NPAD = 10240
RPT = NPAD // NS

CH = 128
EPT_DEG = 10240
NCH = EPT_DEG // CH
E_PAD = NW * EPT_DEG

ACH = 128
EPT = E_PAD // NS
ANCH = EPT // ACH
QTR = ANCH // 4
NBUF = 2

_MESH = plsc.VectorSubcoreMesh(core_axis_name="c", subcore_axis_name="s")


def _deg_body(dstg_hbm, cnt_out, cnt_sh, idxd_v, ones_v, zer_v):
    c = lax.axis_index("c")
    s = lax.axis_index("s")
    w = c * NS + s
    for i in range(8):
        ones_v[pl.ds(i * 16, 16)] = jnp.ones((16,), jnp.float32)
    for i in range(RPT // 16):
        zer_v[pl.ds(i * 16, 16)] = jnp.zeros((16,), jnp.float32)
    pltpu.sync_copy(zer_v, cnt_sh.at[pl.ds(s * RPT, RPT)])
    pltpu.sync_copy(dstg_hbm.at[w], idxd_v)
    plsc.subcore_barrier()

    @pl.loop(0, NCH)
    def _chunk(g):
        pltpu.sync_copy(ones_v, cnt_sh.at[idxd_v.at[g]], add=True)

    plsc.subcore_barrier()
    pltpu.sync_copy(cnt_sh.at[pl.ds(s * RPT, RPT)],
                    cnt_out.at[c, pl.ds(s * RPT, RPT)])


_deg_kernel = pl.kernel(
    _deg_body,
    out_type=jax.ShapeDtypeStruct((NC, NPAD), jnp.float32),
    mesh=_MESH,
    scratch_types=[
        pltpu.VMEM_SHARED((NPAD,), jnp.float32),
        pltpu.VMEM((NCH, CH), jnp.int32),
        pltpu.VMEM((CH,), jnp.float32),
        pltpu.VMEM((RPT,), jnp.float32),
    ],
)


def _agg_body(hs_hbm, srcg_hbm, dstg_hbm, agg_out,
              hs_sh, agg_sh, idxs_v, idxd_v, rows0, rows1,
              sem0, sem1):
    c = lax.axis_index("c")
    s = lax.axis_index("s")
    rows = (rows0, rows1)
    sems = (sem0, sem1)

    @pl.loop(0, ACH)
    def _zr(i):
        @pl.loop(0, FH // 16)
        def _zc(j):
            rows0[i, pl.ds(j * 16, 16)] = jnp.zeros((16,), jnp.float32)

    @pl.loop(0, RPT // ACH)
    def _z(j):
        pltpu.sync_copy(rows0, agg_sh.at[pl.ds(s * RPT + j * ACH, ACH)])
        pltpu.sync_copy(hs_hbm.at[c, pl.ds(s * RPT + j * ACH, ACH)],
                        hs_sh.at[pl.ds(s * RPT + j * ACH, ACH)])

    plsc.subcore_barrier()

    for h in range(ANCH // QTR):
        pltpu.sync_copy(srcg_hbm.at[s, pl.ds(h * QTR, QTR)], idxs_v)
        pltpu.sync_copy(dstg_hbm.at[s, pl.ds(h * QTR, QTR)], idxd_v)
        for b in range(NBUF):
            pltpu.async_copy(hs_sh.at[idxs_v.at[b]], rows[b], sems[b])

        @pl.loop(0, QTR, step=NBUF)
        def _chunks(g):
            for b in range(NBUF):
                pltpu.make_async_copy(hs_sh.at[idxs_v.at[0]],
                                      rows[b], sems[b]).wait()
                pltpu.sync_copy(rows[b], agg_sh.at[idxd_v.at[g + b]],
                                add=True)

                @pl.when(g + b + NBUF < QTR)
                def _(b=b, g=g):
                    pltpu.async_copy(hs_sh.at[idxs_v.at[g + b + NBUF]],
                                     rows[b], sems[b])

    plsc.subcore_barrier()

    @pl.loop(0, RPT // ACH)
    def _wb(j):
        pltpu.sync_copy(agg_sh.at[pl.ds(s * RPT + j * ACH, ACH)], rows0)
        pltpu.sync_copy(rows0, agg_out.at[c, pl.ds(s * RPT + j * ACH, ACH)])


_agg_kernel = pl.kernel(
    _agg_body,
    out_type=jax.ShapeDtypeStruct((NC, NPAD, FH), jnp.float32),
    mesh=_MESH,
    compiler_params=pltpu.CompilerParams(use_tc_tiling_on_sc=False),
    scratch_types=[
        pltpu.VMEM_SHARED((NPAD, FH), jnp.float32),
        pltpu.VMEM_SHARED((NPAD, FH), jnp.float32),
        pltpu.VMEM((QTR, ACH), jnp.int32),
        pltpu.VMEM((QTR, ACH), jnp.int32),
        pltpu.VMEM((ACH, FH), jnp.float32),
        pltpu.VMEM((ACH, FH), jnp.float32),
        pltpu.SemaphoreType.DMA,
        pltpu.SemaphoreType.DMA,
    ],
)


_R = 512


def _dinv_of(cnt_ref):
    return lax.rsqrt(cnt_ref[0, :] + cnt_ref[1, :] + 1.0)


def _split_store(h, out_ref):
    out_ref[0, :, :] = h[:, :FH]
    out_ref[1, :, :] = h[:, FH:]


def _mm_first_body(x_ref, w_ref, cnt_ref, out_ref):
    dinv = _dinv_of(cnt_ref)
    h = jnp.dot(x_ref[...], w_ref[...], preferred_element_type=jnp.float32)
    _split_store(h * dinv[:, None], out_ref)


def _mm_mid_body(agg_ref, hs_ref, cnt_ref, w_ref, b_ref, out_ref, *, leaky):
    dinv = _dinv_of(cnt_ref)
    a = jnp.concatenate([agg_ref[0] + hs_ref[0], agg_ref[1] + hs_ref[1]],
                        axis=-1)
    xn = dinv[:, None] * a + b_ref[...]
    if leaky:
        xn = jnp.where(xn >= 0, xn, 0.01 * xn)
    h = jnp.dot(xn, w_ref[...], preferred_element_type=jnp.float32)
    _split_store(h * dinv[:, None], out_ref)


def _fin_body(agg_ref, hs_ref, cnt_ref, b_ref, out_ref):
    dinv = _dinv_of(cnt_ref)
    a = jnp.concatenate([agg_ref[0] + hs_ref[0], agg_ref[1] + hs_ref[1]],
                        axis=-1)
    xn = dinv[:, None] * a + b_ref[...]
    out_ref[...] = jnp.where(xn >= 0, xn, 0.01 * xn)


_spec_rows = pl.BlockSpec((_R, D), lambda i: (i, 0))
_spec_w = pl.BlockSpec((D, D), lambda i: (0, 0))
_spec_cnt = pl.BlockSpec((NC, _R), lambda i: (0, i))
_spec_half = pl.BlockSpec((NC, _R, FH), lambda i: (0, i, 0))
_spec_b = pl.BlockSpec((1, D), lambda i: (0, 0))
_out_half = jax.ShapeDtypeStruct((NC, NPAD, FH), jnp.float32)
_out_rows = jax.ShapeDtypeStruct((NPAD, D), jnp.float32)

_mm_first = pl.pallas_call(
    _mm_first_body,
    grid=(NPAD // _R,),
    in_specs=[_spec_rows, _spec_w, _spec_cnt],
    out_specs=_spec_half,
    out_shape=_out_half,
)

_mm_mid_leaky = pl.pallas_call(
    functools.partial(_mm_mid_body, leaky=True),
    grid=(NPAD // _R,),
    in_specs=[_spec_half, _spec_half, _spec_cnt, _spec_w, _spec_b],
    out_specs=_spec_half,
    out_shape=_out_half,
)

_mm_mid_plain = pl.pallas_call(
    functools.partial(_mm_mid_body, leaky=False),
    grid=(NPAD // _R,),
    in_specs=[_spec_half, _spec_half, _spec_cnt, _spec_w, _spec_b],
    out_specs=_spec_half,
    out_shape=_out_half,
)

_fin = pl.pallas_call(
    _fin_body,
    grid=(NPAD // _R,),
    in_specs=[_spec_half, _spec_half, _spec_cnt, _spec_b],
    out_specs=_spec_rows,
    out_shape=_out_rows,
)


def kernel(x, edge_index, W1, b1, W2, b2, W3, b3):
    ei = edge_index.astype(jnp.int32)
    pad = jnp.full((E_PAD - E,), N, jnp.int32)
    src_flat = jnp.concatenate([ei[0], pad])
    dst_flat = jnp.concatenate([ei[1], pad])
    srcg = src_flat.reshape(NS, ANCH, ACH)
    dstg = dst_flat.reshape(NS, ANCH, ACH)
    xp = jnp.pad(x, ((0, NPAD - N), (0, 0)))

    cnt = _deg_kernel(dst_flat.reshape(NW, NCH, CH))

    hs = _mm_first(xp, W1, cnt)
    agg = _agg_kernel(hs, srcg, dstg)
    hs = _mm_mid_leaky(agg, hs, cnt, W2, b1.reshape(1, D))
    agg = _agg_kernel(hs, srcg, dstg)
    hs = _mm_mid_plain(agg, hs, cnt, W3, b2.reshape(1, D))
    agg = _agg_kernel(hs, srcg, dstg)
    out = _fin(agg, hs, cnt, b3.reshape(1, D))
    return out[:N]


# speedup vs baseline: 1.0278x; 1.0278x over previous
import functools

import jax
import jax.numpy as jnp
from jax import lax
from jax.experimental import pallas as pl
from jax.experimental.pallas import tpu as pltpu
from jax.experimental.pallas import tpu_sc as plsc

N = 10000
D = 128
FH = 64
E = 320000
NC = 2
NS = 16
NW = NC * NS
NPAD = 10240
RPT = NPAD // NS

CH = 128
EPT_DEG = 10240
NCH = EPT_DEG // CH
E_PAD = NW * EPT_DEG

ACH = 64
EPT = E_PAD // NS
ANCH = EPT // ACH
QTR = ANCH // 4
NBUF = 2

_MESH = plsc.VectorSubcoreMesh(core_axis_name="c", subcore_axis_name="s")


def _deg_body(dstg_hbm, cnt_out, cnt_sh, idxd_v, ones_v, zer_v):
    c = lax.axis_index("c")
    s = lax.axis_index("s")
    w = c * NS + s
    for i in range(8):
        ones_v[pl.ds(i * 16, 16)] = jnp.ones((16,), jnp.float32)
    for i in range(RPT // 16):
        zer_v[pl.ds(i * 16, 16)] = jnp.zeros((16,), jnp.float32)
    pltpu.sync_copy(zer_v, cnt_sh.at[pl.ds(s * RPT, RPT)])
    pltpu.sync_copy(dstg_hbm.at[w], idxd_v)
    plsc.subcore_barrier()

    @pl.loop(0, NCH)
    def _chunk(g):
        pltpu.sync_copy(ones_v, cnt_sh.at[idxd_v.at[g]], add=True)

    plsc.subcore_barrier()
    pltpu.sync_copy(cnt_sh.at[pl.ds(s * RPT, RPT)],
                    cnt_out.at[c, pl.ds(s * RPT, RPT)])


_deg_kernel = pl.kernel(
    _deg_body,
    out_type=jax.ShapeDtypeStruct((NC, NPAD), jnp.float32),
    mesh=_MESH,
    scratch_types=[
        pltpu.VMEM_SHARED((NPAD,), jnp.float32),
        pltpu.VMEM((NCH, CH), jnp.int32),
        pltpu.VMEM((CH,), jnp.float32),
        pltpu.VMEM((RPT,), jnp.float32),
    ],
)


def _agg_body(hs_hbm, srcg_hbm, dstg_hbm, agg_out,
              hs_sh, agg_sh, idxs_v, idxd_v, rows0, rows1,
              sem0, sem1):
    c = lax.axis_index("c")
    s = lax.axis_index("s")
    rows = (rows0, rows1)
    sems = (sem0, sem1)

    @pl.loop(0, ACH)
    def _zr(i):
        @pl.loop(0, FH // 16)
        def _zc(j):
            rows0[i, pl.ds(j * 16, 16)] = jnp.zeros((16,), jnp.float32)

    @pl.loop(0, RPT // ACH)
    def _z(j):
        pltpu.sync_copy(rows0, agg_sh.at[pl.ds(s * RPT + j * ACH, ACH)])
        pltpu.sync_copy(hs_hbm.at[c, pl.ds(s * RPT + j * ACH, ACH)],
                        hs_sh.at[pl.ds(s * RPT + j * ACH, ACH)])

    plsc.subcore_barrier()

    for h in range(ANCH // QTR):
        pltpu.sync_copy(srcg_hbm.at[s, pl.ds(h * QTR, QTR)], idxs_v)
        pltpu.sync_copy(dstg_hbm.at[s, pl.ds(h * QTR, QTR)], idxd_v)
        for b in range(NBUF):
            pltpu.async_copy(hs_sh.at[idxs_v.at[b]], rows[b], sems[b])

        @pl.loop(0, QTR, step=NBUF)
        def _chunks(g):
            for b in range(NBUF):
                pltpu.make_async_copy(hs_sh.at[idxs_v.at[0]],
                                      rows[b], sems[b]).wait()
                pltpu.sync_copy(rows[b], agg_sh.at[idxd_v.at[g + b]],
                                add=True)

                @pl.when(g + b + NBUF < QTR)
                def _(b=b, g=g):
                    pltpu.async_copy(hs_sh.at[idxs_v.at[g + b + NBUF]],
                                     rows[b], sems[b])

    plsc.subcore_barrier()

    @pl.loop(0, RPT // ACH)
    def _wb(j):
        pltpu.sync_copy(agg_sh.at[pl.ds(s * RPT + j * ACH, ACH)], rows0)
        pltpu.sync_copy(rows0, agg_out.at[c, pl.ds(s * RPT + j * ACH, ACH)])


_agg_kernel = pl.kernel(
    _agg_body,
    out_type=jax.ShapeDtypeStruct((NC, NPAD, FH), jnp.float32),
    mesh=_MESH,
    compiler_params=pltpu.CompilerParams(use_tc_tiling_on_sc=False),
    scratch_types=[
        pltpu.VMEM_SHARED((NPAD, FH), jnp.float32),
        pltpu.VMEM_SHARED((NPAD, FH), jnp.float32),
        pltpu.VMEM((QTR, ACH), jnp.int32),
        pltpu.VMEM((QTR, ACH), jnp.int32),
        pltpu.VMEM((ACH, FH), jnp.float32),
        pltpu.VMEM((ACH, FH), jnp.float32),
        pltpu.SemaphoreType.DMA,
        pltpu.SemaphoreType.DMA,
    ],
)


_R = 512


def _dinv_of(cnt_ref):
    return lax.rsqrt(cnt_ref[0, :] + cnt_ref[1, :] + 1.0)


def _split_store(h, out_ref):
    out_ref[0, :, :] = h[:, :FH]
    out_ref[1, :, :] = h[:, FH:]


def _mm_first_body(x_ref, w_ref, cnt_ref, out_ref):
    dinv = _dinv_of(cnt_ref)
    h = jnp.dot(x_ref[...], w_ref[...], preferred_element_type=jnp.float32)
    _split_store(h * dinv[:, None], out_ref)


def _mm_mid_body(agg_ref, hs_ref, cnt_ref, w_ref, b_ref, out_ref, *, leaky):
    dinv = _dinv_of(cnt_ref)
    a = jnp.concatenate([agg_ref[0] + hs_ref[0], agg_ref[1] + hs_ref[1]],
                        axis=-1)
    xn = dinv[:, None] * a + b_ref[...]
    if leaky:
        xn = jnp.where(xn >= 0, xn, 0.01 * xn)
    h = jnp.dot(xn, w_ref[...], preferred_element_type=jnp.float32)
    _split_store(h * dinv[:, None], out_ref)


def _fin_body(agg_ref, hs_ref, cnt_ref, b_ref, out_ref):
    dinv = _dinv_of(cnt_ref)
    a = jnp.concatenate([agg_ref[0] + hs_ref[0], agg_ref[1] + hs_ref[1]],
                        axis=-1)
    xn = dinv[:, None] * a + b_ref[...]
    out_ref[...] = jnp.where(xn >= 0, xn, 0.01 * xn)


_spec_rows = pl.BlockSpec((_R, D), lambda i: (i, 0))
_spec_w = pl.BlockSpec((D, D), lambda i: (0, 0))
_spec_cnt = pl.BlockSpec((NC, _R), lambda i: (0, i))
_spec_half = pl.BlockSpec((NC, _R, FH), lambda i: (0, i, 0))
_spec_b = pl.BlockSpec((1, D), lambda i: (0, 0))
_out_half = jax.ShapeDtypeStruct((NC, NPAD, FH), jnp.float32)
_out_rows = jax.ShapeDtypeStruct((NPAD, D), jnp.float32)

_mm_first = pl.pallas_call(
    _mm_first_body,
    grid=(NPAD // _R,),
    in_specs=[_spec_rows, _spec_w, _spec_cnt],
    out_specs=_spec_half,
    out_shape=_out_half,
)

_mm_mid_leaky = pl.pallas_call(
    functools.partial(_mm_mid_body, leaky=True),
    grid=(NPAD // _R,),
    in_specs=[_spec_half, _spec_half, _spec_cnt, _spec_w, _spec_b],
    out_specs=_spec_half,
    out_shape=_out_half,
)

_mm_mid_plain = pl.pallas_call(
    functools.partial(_mm_mid_body, leaky=False),
    grid=(NPAD // _R,),
    in_specs=[_spec_half, _spec_half, _spec_cnt, _spec_w, _spec_b],
    out_specs=_spec_half,
    out_shape=_out_half,
)

_fin = pl.pallas_call(
    _fin_body,
    grid=(NPAD // _R,),
    in_specs=[_spec_half, _spec_half, _spec_cnt, _spec_b],
    out_specs=_spec_rows,
    out_shape=_out_rows,
)


def kernel(x, edge_index, W1, b1, W2, b2, W3, b3):
    ei = edge_index.astype(jnp.int32)
    pad = jnp.full((E_PAD - E,), N, jnp.int32)
    src_flat = jnp.concatenate([ei[0], pad])
    dst_flat = jnp.concatenate([ei[1], pad])
    srcg = src_flat.reshape(NS, ANCH, ACH)
    dstg = dst_flat.reshape(NS, ANCH, ACH)
    xp = jnp.pad(x, ((0, NPAD - N), (0, 0)))

    cnt = _deg_kernel(dst_flat.reshape(NW, NCH, CH))

    hs = _mm_first(xp, W1, cnt)
    agg = _agg_kernel(hs, srcg, dstg)
    hs = _mm_mid_leaky(agg, hs, cnt, W2, b1.reshape(1, D))
    agg = _agg_kernel(hs, srcg, dstg)
    hs = _mm_mid_plain(agg, hs, cnt, W3, b2.reshape(1, D))
    agg = _agg_kernel(hs, srcg, dstg)
    out = _fin(agg, hs, cnt, b3.reshape(1, D))
    return out[:N]


# speedup vs baseline: 1.1504x; 1.1192x over previous
import functools

import jax
import jax.numpy as jnp
from jax import lax
from jax.experimental import pallas as pl
from jax.experimental.pallas import tpu as pltpu
from jax.experimental.pallas import tpu_sc as plsc

N = 10000
D = 128
FH = 64
E = 320000
NC = 2
NS = 16
NW = NC * NS
NPAD = 10240
RPT = NPAD // NS

CH = 128
EPT_DEG = 10240
NCH = EPT_DEG // CH
E_PAD = NW * EPT_DEG

ACH = 64
EPT = E_PAD // NS
ANCH = EPT // ACH
QTR = ANCH // 4
NBUF = 2

_MESH = plsc.VectorSubcoreMesh(core_axis_name="c", subcore_axis_name="s")


def _deg_body(dstg_hbm, cnt_out, cnt_sh, idxd_v, ones_v, zer_v):
    c = lax.axis_index("c")
    s = lax.axis_index("s")
    w = c * NS + s
    for i in range(8):
        ones_v[pl.ds(i * 16, 16)] = jnp.ones((16,), jnp.float32)
    for i in range(RPT // 16):
        zer_v[pl.ds(i * 16, 16)] = jnp.zeros((16,), jnp.float32)
    pltpu.sync_copy(zer_v, cnt_sh.at[pl.ds(s * RPT, RPT)])
    pltpu.sync_copy(dstg_hbm.at[w], idxd_v)
    plsc.subcore_barrier()

    @pl.loop(0, NCH)
    def _chunk(g):
        pltpu.sync_copy(ones_v, cnt_sh.at[idxd_v.at[g]], add=True)

    plsc.subcore_barrier()
    pltpu.sync_copy(cnt_sh.at[pl.ds(s * RPT, RPT)],
                    cnt_out.at[c, pl.ds(s * RPT, RPT)])


_deg_kernel = pl.kernel(
    _deg_body,
    out_type=jax.ShapeDtypeStruct((NC, NPAD), jnp.float32),
    mesh=_MESH,
    scratch_types=[
        pltpu.VMEM_SHARED((NPAD,), jnp.float32),
        pltpu.VMEM((NCH, CH), jnp.int32),
        pltpu.VMEM((CH,), jnp.float32),
        pltpu.VMEM((RPT,), jnp.float32),
    ],
)


def _agg_body(hs_hbm, srcg_hbm, dstg_hbm, agg_out,
              hs_sh, agg_sh, idxs_v, idxd_v, rows0, rows1,
              sem0, sem1):
    c = lax.axis_index("c")
    s = lax.axis_index("s")
    rows = (rows0, rows1)
    sems = (sem0, sem1)

    @pl.loop(0, ACH)
    def _zr(i):
        @pl.loop(0, FH // 16)
        def _zc(j):
            rows0[i, pl.ds(j * 16, 16)] = jnp.zeros((16,), jnp.float32)

    @pl.loop(0, RPT // ACH)
    def _z(j):
        pltpu.sync_copy(rows0, agg_sh.at[pl.ds(s * RPT + j * ACH, ACH)])
        pltpu.sync_copy(hs_hbm.at[pl.ds(s * RPT + j * ACH, ACH),
                                  pl.ds(c * FH, FH)],
                        hs_sh.at[pl.ds(s * RPT + j * ACH, ACH)])

    plsc.subcore_barrier()

    for h in range(ANCH // QTR):
        pltpu.sync_copy(srcg_hbm.at[s, pl.ds(h * QTR, QTR)], idxs_v)
        pltpu.sync_copy(dstg_hbm.at[s, pl.ds(h * QTR, QTR)], idxd_v)
        for b in range(NBUF):
            pltpu.async_copy(hs_sh.at[idxs_v.at[b]], rows[b], sems[b])

        @pl.loop(0, QTR, step=NBUF)
        def _chunks(g):
            for b in range(NBUF):
                pltpu.make_async_copy(hs_sh.at[idxs_v.at[0]],
                                      rows[b], sems[b]).wait()
                pltpu.sync_copy(rows[b], agg_sh.at[idxd_v.at[g + b]],
                                add=True)

                @pl.when(g + b + NBUF < QTR)
                def _(b=b, g=g):
                    pltpu.async_copy(hs_sh.at[idxs_v.at[g + b + NBUF]],
                                     rows[b], sems[b])

    plsc.subcore_barrier()

    @pl.loop(0, RPT // ACH)
    def _wb(j):
        pltpu.sync_copy(agg_sh.at[pl.ds(s * RPT + j * ACH, ACH)], rows0)
        pltpu.sync_copy(rows0, agg_out.at[pl.ds(s * RPT + j * ACH, ACH),
                                          pl.ds(c * FH, FH)])


_agg_kernel = pl.kernel(
    _agg_body,
    out_type=jax.ShapeDtypeStruct((NPAD, D), jnp.float32),
    mesh=_MESH,
    compiler_params=pltpu.CompilerParams(use_tc_tiling_on_sc=False),
    scratch_types=[
        pltpu.VMEM_SHARED((NPAD, FH), jnp.float32),
        pltpu.VMEM_SHARED((NPAD, FH), jnp.float32),
        pltpu.VMEM((QTR, ACH), jnp.int32),
        pltpu.VMEM((QTR, ACH), jnp.int32),
        pltpu.VMEM((ACH, FH), jnp.float32),
        pltpu.VMEM((ACH, FH), jnp.float32),
        pltpu.SemaphoreType.DMA,
        pltpu.SemaphoreType.DMA,
    ],
)


_R = 512


def _dinv_of(cnt_ref):
    return lax.rsqrt(cnt_ref[0, :] + cnt_ref[1, :] + 1.0)


def _mm_first_body(x_ref, w_ref, cnt_ref, out_ref):
    dinv = _dinv_of(cnt_ref)
    h = jnp.dot(x_ref[...], w_ref[...], preferred_element_type=jnp.float32)
    out_ref[...] = h * dinv[:, None]


def _mm_mid_body(agg_ref, hs_ref, cnt_ref, w_ref, b_ref, out_ref, *, leaky):
    dinv = _dinv_of(cnt_ref)
    a = agg_ref[...] + hs_ref[...]
    xn = dinv[:, None] * a + b_ref[...]
    if leaky:
        xn = jnp.where(xn >= 0, xn, 0.01 * xn)
    h = jnp.dot(xn, w_ref[...], preferred_element_type=jnp.float32)
    out_ref[...] = h * dinv[:, None]


def _fin_body(agg_ref, hs_ref, cnt_ref, b_ref, out_ref):
    dinv = _dinv_of(cnt_ref)
    a = agg_ref[...] + hs_ref[...]
    xn = dinv[:, None] * a + b_ref[...]
    out_ref[...] = jnp.where(xn >= 0, xn, 0.01 * xn)


_spec_rows = pl.BlockSpec((_R, D), lambda i: (i, 0))
_spec_w = pl.BlockSpec((D, D), lambda i: (0, 0))
_spec_cnt = pl.BlockSpec((NC, _R), lambda i: (0, i))
_spec_b = pl.BlockSpec((1, D), lambda i: (0, 0))
_out_rows = jax.ShapeDtypeStruct((NPAD, D), jnp.float32)

_mm_first = pl.pallas_call(
    _mm_first_body,
    grid=(NPAD // _R,),
    in_specs=[_spec_rows, _spec_w, _spec_cnt],
    out_specs=_spec_rows,
    out_shape=_out_rows,
)

_mm_mid_leaky = pl.pallas_call(
    functools.partial(_mm_mid_body, leaky=True),
    grid=(NPAD // _R,),
    in_specs=[_spec_rows, _spec_rows, _spec_cnt, _spec_w, _spec_b],
    out_specs=_spec_rows,
    out_shape=_out_rows,
)

_mm_mid_plain = pl.pallas_call(
    functools.partial(_mm_mid_body, leaky=False),
    grid=(NPAD // _R,),
    in_specs=[_spec_rows, _spec_rows, _spec_cnt, _spec_w, _spec_b],
    out_specs=_spec_rows,
    out_shape=_out_rows,
)

_fin = pl.pallas_call(
    _fin_body,
    grid=(NPAD // _R,),
    in_specs=[_spec_rows, _spec_rows, _spec_cnt, _spec_b],
    out_specs=_spec_rows,
    out_shape=_out_rows,
)


def kernel(x, edge_index, W1, b1, W2, b2, W3, b3):
    ei = edge_index.astype(jnp.int32)
    pad = jnp.full((E_PAD - E,), N, jnp.int32)
    src_flat = jnp.concatenate([ei[0], pad])
    dst_flat = jnp.concatenate([ei[1], pad])
    srcg = src_flat.reshape(NS, ANCH, ACH)
    dstg = dst_flat.reshape(NS, ANCH, ACH)
    xp = jnp.pad(x, ((0, NPAD - N), (0, 0)))

    cnt = _deg_kernel(dst_flat.reshape(NW, NCH, CH))

    hs = _mm_first(xp, W1, cnt)
    agg = _agg_kernel(hs, srcg, dstg)
    hs = _mm_mid_leaky(agg, hs, cnt, W2, b1.reshape(1, D))
    agg = _agg_kernel(hs, srcg, dstg)
    hs = _mm_mid_plain(agg, hs, cnt, W3, b2.reshape(1, D))
    agg = _agg_kernel(hs, srcg, dstg)
    out = _fin(agg, hs, cnt, b3.reshape(1, D))
    return out[:N]


# speedup vs baseline: 1.1511x; 1.0006x over previous
import functools

import jax
import jax.numpy as jnp
from jax import lax
from jax.experimental import pallas as pl
from jax.experimental.pallas import tpu as pltpu
from jax.experimental.pallas import tpu_sc as plsc

N = 10000
D = 128
FH = 64
E = 320000
NC = 2
NS = 16
NW = NC * NS
NPAD = 10240
RPT = NPAD // NS

CH = 128
EPT_DEG = 10240
NCH = EPT_DEG // CH
E_PAD = NW * EPT_DEG

ACH = 64
EPT = E_PAD // NS
ANCH = EPT // ACH
QTR = ANCH // 4
NBUF = 2

_MESH = plsc.VectorSubcoreMesh(core_axis_name="c", subcore_axis_name="s")


def _deg_body(dstg_hbm, cnt_out, cnt_sh, idxd_v, ones_v, zer_v):
    c = lax.axis_index("c")
    s = lax.axis_index("s")
    w = c * NS + s
    for i in range(8):
        ones_v[pl.ds(i * 16, 16)] = jnp.ones((16,), jnp.float32)
    for i in range(RPT // 16):
        zer_v[pl.ds(i * 16, 16)] = jnp.zeros((16,), jnp.float32)
    pltpu.sync_copy(zer_v, cnt_sh.at[pl.ds(s * RPT, RPT)])
    pltpu.sync_copy(dstg_hbm.at[w], idxd_v)
    plsc.subcore_barrier()

    @pl.loop(0, NCH)
    def _chunk(g):
        pltpu.sync_copy(ones_v, cnt_sh.at[idxd_v.at[g]], add=True)

    plsc.subcore_barrier()
    pltpu.sync_copy(cnt_sh.at[pl.ds(s * RPT, RPT)],
                    cnt_out.at[c, pl.ds(s * RPT, RPT)])


_deg_kernel = pl.kernel(
    _deg_body,
    out_type=jax.ShapeDtypeStruct((NC, NPAD), jnp.float32),
    mesh=_MESH,
    scratch_types=[
        pltpu.VMEM_SHARED((NPAD,), jnp.float32),
        pltpu.VMEM((NCH, CH), jnp.int32),
        pltpu.VMEM((CH,), jnp.float32),
        pltpu.VMEM((RPT,), jnp.float32),
    ],
)


def _agg_body(hs_hbm, srcg_hbm, dstg_hbm, agg_out,
              hs_sh, agg_sh, idxs_v, idxd_v, rows0, rows1,
              sem0, sem1, ssem0, ssem1):
    c = lax.axis_index("c")
    s = lax.axis_index("s")
    rows = (rows0, rows1)
    sems = (sem0, sem1)
    ssems = (ssem0, ssem1)

    @pl.loop(0, ACH)
    def _zr(i):
        @pl.loop(0, FH // 16)
        def _zc(j):
            rows0[i, pl.ds(j * 16, 16)] = jnp.zeros((16,), jnp.float32)

    @pl.loop(0, RPT // ACH)
    def _z(j):
        pltpu.sync_copy(rows0, agg_sh.at[pl.ds(s * RPT + j * ACH, ACH)])
        pltpu.sync_copy(hs_hbm.at[pl.ds(s * RPT + j * ACH, ACH),
                                  pl.ds(c * FH, FH)],
                        hs_sh.at[pl.ds(s * RPT + j * ACH, ACH)])

    plsc.subcore_barrier()

    for h in range(ANCH // QTR):
        pltpu.sync_copy(srcg_hbm.at[s, pl.ds(h * QTR, QTR)], idxs_v)
        pltpu.sync_copy(dstg_hbm.at[s, pl.ds(h * QTR, QTR)], idxd_v)
        for b in range(NBUF):
            pltpu.async_copy(hs_sh.at[idxs_v.at[b]], rows[b], sems[b])

        @pl.loop(0, QTR, step=NBUF)
        def _chunks(g):
            for b in range(NBUF):
                pltpu.make_async_copy(hs_sh.at[idxs_v.at[0]],
                                      rows[b], sems[b]).wait()
                pltpu.async_copy(rows[b], agg_sh.at[idxd_v.at[g + b]],
                                 ssems[b], add=True)

                @pl.when(g + b + NBUF < QTR)
                def _(b=b, g=g):
                    pltpu.make_async_copy(rows[b], agg_sh.at[idxd_v.at[0]],
                                          ssems[b]).wait()
                    pltpu.async_copy(hs_sh.at[idxs_v.at[g + b + NBUF]],
                                     rows[b], sems[b])

        for b in range(NBUF):
            @pl.when(QTR - NBUF + b >= 0)
            def _(b=b):
                pltpu.make_async_copy(rows[b], agg_sh.at[idxd_v.at[0]],
                                      ssems[b]).wait()

    plsc.subcore_barrier()

    @pl.loop(0, RPT // ACH)
    def _wb(j):
        pltpu.sync_copy(agg_sh.at[pl.ds(s * RPT + j * ACH, ACH)], rows0)
        pltpu.sync_copy(rows0, agg_out.at[pl.ds(s * RPT + j * ACH, ACH),
                                          pl.ds(c * FH, FH)])


_agg_kernel = pl.kernel(
    _agg_body,
    out_type=jax.ShapeDtypeStruct((NPAD, D), jnp.float32),
    mesh=_MESH,
    compiler_params=pltpu.CompilerParams(use_tc_tiling_on_sc=False),
    scratch_types=[
        pltpu.VMEM_SHARED((NPAD, FH), jnp.float32),
        pltpu.VMEM_SHARED((NPAD, FH), jnp.float32),
        pltpu.VMEM((QTR, ACH), jnp.int32),
        pltpu.VMEM((QTR, ACH), jnp.int32),
        pltpu.VMEM((ACH, FH), jnp.float32),
        pltpu.VMEM((ACH, FH), jnp.float32),
        pltpu.SemaphoreType.DMA,
        pltpu.SemaphoreType.DMA,
        pltpu.SemaphoreType.DMA,
        pltpu.SemaphoreType.DMA,
    ],
)


_R = 512


def _dinv_of(cnt_ref):
    return lax.rsqrt(cnt_ref[0, :] + cnt_ref[1, :] + 1.0)


def _mm_first_body(x_ref, w_ref, cnt_ref, out_ref):
    dinv = _dinv_of(cnt_ref)
    h = jnp.dot(x_ref[...], w_ref[...], preferred_element_type=jnp.float32)
    out_ref[...] = h * dinv[:, None]


def _mm_mid_body(agg_ref, hs_ref, cnt_ref, w_ref, b_ref, out_ref, *, leaky):
    dinv = _dinv_of(cnt_ref)
    a = agg_ref[...] + hs_ref[...]
    xn = dinv[:, None] * a + b_ref[...]
    if leaky:
        xn = jnp.where(xn >= 0, xn, 0.01 * xn)
    h = jnp.dot(xn, w_ref[...], preferred_element_type=jnp.float32)
    out_ref[...] = h * dinv[:, None]


def _fin_body(agg_ref, hs_ref, cnt_ref, b_ref, out_ref):
    dinv = _dinv_of(cnt_ref)
    a = agg_ref[...] + hs_ref[...]
    xn = dinv[:, None] * a + b_ref[...]
    out_ref[...] = jnp.where(xn >= 0, xn, 0.01 * xn)


_spec_rows = pl.BlockSpec((_R, D), lambda i: (i, 0))
_spec_w = pl.BlockSpec((D, D), lambda i: (0, 0))
_spec_cnt = pl.BlockSpec((NC, _R), lambda i: (0, i))
_spec_b = pl.BlockSpec((1, D), lambda i: (0, 0))
_out_rows = jax.ShapeDtypeStruct((NPAD, D), jnp.float32)

_mm_first = pl.pallas_call(
    _mm_first_body,
    grid=(NPAD // _R,),
    in_specs=[_spec_rows, _spec_w, _spec_cnt],
    out_specs=_spec_rows,
    out_shape=_out_rows,
)

_mm_mid_leaky = pl.pallas_call(
    functools.partial(_mm_mid_body, leaky=True),
    grid=(NPAD // _R,),
    in_specs=[_spec_rows, _spec_rows, _spec_cnt, _spec_w, _spec_b],
    out_specs=_spec_rows,
    out_shape=_out_rows,
)

_mm_mid_plain = pl.pallas_call(
    functools.partial(_mm_mid_body, leaky=False),
    grid=(NPAD // _R,),
    in_specs=[_spec_rows, _spec_rows, _spec_cnt, _spec_w, _spec_b],
    out_specs=_spec_rows,
    out_shape=_out_rows,
)

_fin = pl.pallas_call(
    _fin_body,
    grid=(NPAD // _R,),
    in_specs=[_spec_rows, _spec_rows, _spec_cnt, _spec_b],
    out_specs=_spec_rows,
    out_shape=_out_rows,
)


def kernel(x, edge_index, W1, b1, W2, b2, W3, b3):
    ei = edge_index.astype(jnp.int32)
    pad = jnp.full((E_PAD - E,), N, jnp.int32)
    src_flat = jnp.concatenate([ei[0], pad])
    dst_flat = jnp.concatenate([ei[1], pad])
    srcg = src_flat.reshape(NS, ANCH, ACH)
    dstg = dst_flat.reshape(NS, ANCH, ACH)
    xp = jnp.pad(x, ((0, NPAD - N), (0, 0)))

    cnt = _deg_kernel(dst_flat.reshape(NW, NCH, CH))

    hs = _mm_first(xp, W1, cnt)
    agg = _agg_kernel(hs, srcg, dstg)
    hs = _mm_mid_leaky(agg, hs, cnt, W2, b1.reshape(1, D))
    agg = _agg_kernel(hs, srcg, dstg)
    hs = _mm_mid_plain(agg, hs, cnt, W3, b2.reshape(1, D))
    agg = _agg_kernel(hs, srcg, dstg)
    out = _fin(agg, hs, cnt, b3.reshape(1, D))
    return out[:N]


# speedup vs baseline: 1.1928x; 1.0362x over previous
import functools

import jax
import jax.numpy as jnp
from jax import lax
from jax.experimental import pallas as pl
from jax.experimental.pallas import tpu as pltpu
from jax.experimental.pallas import tpu_sc as plsc

N = 10000
D = 128
FH = 64
E = 320000
NC = 2
NS = 16
NW = NC * NS
NPAD = 10240
RPT = NPAD // NS

CH = 128
EPT_DEG = 10240
NCH = EPT_DEG // CH
E_PAD = NW * EPT_DEG

ACH = 64
EPT = E_PAD // NS
ANCH = EPT // ACH
QTR = ANCH // 4
NBUF = 2

_MESH = plsc.VectorSubcoreMesh(core_axis_name="c", subcore_axis_name="s")


def _deg_body(dstg_hbm, cnt_out, cnt_sh, idxd_v, ones_v, zer_v):
    c = lax.axis_index("c")
    s = lax.axis_index("s")
    w = c * NS + s
    for i in range(8):
        ones_v[pl.ds(i * 16, 16)] = jnp.ones((16,), jnp.float32)
    for i in range(RPT // 16):
        zer_v[pl.ds(i * 16, 16)] = jnp.zeros((16,), jnp.float32)
    pltpu.sync_copy(zer_v, cnt_sh.at[pl.ds(s * RPT, RPT)])
    pltpu.sync_copy(dstg_hbm.at[w], idxd_v)
    plsc.subcore_barrier()

    @pl.loop(0, NCH)
    def _chunk(g):
        pltpu.sync_copy(ones_v, cnt_sh.at[idxd_v.at[g]], add=True)

    plsc.subcore_barrier()
    pltpu.sync_copy(cnt_sh.at[pl.ds(s * RPT, RPT)],
                    cnt_out.at[c, pl.ds(s * RPT, RPT)])


_deg_kernel = pl.kernel(
    _deg_body,
    out_type=jax.ShapeDtypeStruct((NC, NPAD), jnp.float32),
    mesh=_MESH,
    scratch_types=[
        pltpu.VMEM_SHARED((NPAD,), jnp.float32),
        pltpu.VMEM((NCH, CH), jnp.int32),
        pltpu.VMEM((CH,), jnp.float32),
        pltpu.VMEM((RPT,), jnp.float32),
    ],
)


def _agg_body(hs_hbm, srcg_hbm, dstg_hbm, agg_out,
              hs_sh, agg_sh, idxs_v, idxd_v, rows0, rows1,
              sem0, sem1, ssem0, ssem1):
    c = lax.axis_index("c")
    s = lax.axis_index("s")
    rows = (rows0, rows1)
    sems = (sem0, sem1)
    ssems = (ssem0, ssem1)

    @pl.loop(0, ACH)
    def _zr(i):
        @pl.loop(0, FH // 16)
        def _zc(j):
            rows0[i, pl.ds(j * 16, 16)] = jnp.zeros((16,), jnp.float32)

    @pl.loop(0, RPT // ACH)
    def _z(j):
        pltpu.sync_copy(rows0, agg_sh.at[pl.ds(s * RPT + j * ACH, ACH)])
        pltpu.sync_copy(hs_hbm.at[pl.ds(s * RPT + j * ACH, ACH),
                                  pl.ds(c * FH, FH)],
                        hs_sh.at[pl.ds(s * RPT + j * ACH, ACH)])

    plsc.subcore_barrier()

    for h in range(ANCH // QTR):
        pltpu.sync_copy(srcg_hbm.at[s, pl.ds(h * QTR, QTR)], idxs_v)
        pltpu.sync_copy(dstg_hbm.at[s, pl.ds(h * QTR, QTR)], idxd_v)
        for b in range(NBUF):
            pltpu.async_copy(hs_sh.at[idxs_v.at[b]], rows[b], sems[b])

        @pl.loop(0, QTR, step=NBUF)
        def _chunks(g):
            for b in range(NBUF):
                pltpu.make_async_copy(hs_sh.at[idxs_v.at[0]],
                                      rows[b], sems[b]).wait()
                pltpu.async_copy(rows[b], agg_sh.at[idxd_v.at[g + b]],
                                 ssems[b], add=True)

                @pl.when(g + b + NBUF < QTR)
                def _(b=b, g=g):
                    pltpu.make_async_copy(rows[b], agg_sh.at[idxd_v.at[0]],
                                          ssems[b]).wait()
                    pltpu.async_copy(hs_sh.at[idxs_v.at[g + b + NBUF]],
                                     rows[b], sems[b])

        for b in range(NBUF):
            @pl.when(QTR - NBUF + b >= 0)
            def _(b=b):
                pltpu.make_async_copy(rows[b], agg_sh.at[idxd_v.at[0]],
                                      ssems[b]).wait()

    plsc.subcore_barrier()

    @pl.loop(0, RPT // ACH)
    def _wb(j):
        pltpu.sync_copy(agg_sh.at[pl.ds(s * RPT + j * ACH, ACH)], rows0)
        pltpu.sync_copy(rows0, agg_out.at[pl.ds(s * RPT + j * ACH, ACH),
                                          pl.ds(c * FH, FH)])


_agg_kernel = pl.kernel(
    _agg_body,
    out_type=jax.ShapeDtypeStruct((NPAD, D), jnp.float32),
    mesh=_MESH,
    compiler_params=pltpu.CompilerParams(use_tc_tiling_on_sc=False),
    scratch_types=[
        pltpu.VMEM_SHARED((NPAD, FH), jnp.float32),
        pltpu.VMEM_SHARED((NPAD, FH), jnp.float32),
        pltpu.VMEM((QTR, ACH), jnp.int32),
        pltpu.VMEM((QTR, ACH), jnp.int32),
        pltpu.VMEM((ACH, FH), jnp.float32),
        pltpu.VMEM((ACH, FH), jnp.float32),
        pltpu.SemaphoreType.DMA,
        pltpu.SemaphoreType.DMA,
        pltpu.SemaphoreType.DMA,
        pltpu.SemaphoreType.DMA,
    ],
)


_R = 1024


def _dinv_of(cnt_ref):
    return lax.rsqrt(cnt_ref[0, :] + cnt_ref[1, :] + 1.0)


def _mm_first_body(x_ref, w_ref, cnt_ref, out_ref):
    dinv = _dinv_of(cnt_ref)
    h = jnp.dot(x_ref[...], w_ref[...], preferred_element_type=jnp.float32)
    out_ref[...] = h * dinv[:, None]


def _mm_mid_body(agg_ref, hs_ref, cnt_ref, w_ref, b_ref, out_ref, *, leaky):
    dinv = _dinv_of(cnt_ref)
    a = agg_ref[...] + hs_ref[...]
    xn = dinv[:, None] * a + b_ref[...]
    if leaky:
        xn = jnp.where(xn >= 0, xn, 0.01 * xn)
    h = jnp.dot(xn, w_ref[...], preferred_element_type=jnp.float32)
    out_ref[...] = h * dinv[:, None]


def _fin_body(agg_ref, hs_ref, cnt_ref, b_ref, out_ref):
    dinv = _dinv_of(cnt_ref)
    a = agg_ref[...] + hs_ref[...]
    xn = dinv[:, None] * a + b_ref[...]
    out_ref[...] = jnp.where(xn >= 0, xn, 0.01 * xn)


_spec_rows = pl.BlockSpec((_R, D), lambda i: (i, 0))
_spec_w = pl.BlockSpec((D, D), lambda i: (0, 0))
_spec_cnt = pl.BlockSpec((NC, _R), lambda i: (0, i))
_spec_b = pl.BlockSpec((1, D), lambda i: (0, 0))
_out_rows = jax.ShapeDtypeStruct((NPAD, D), jnp.float32)

_mm_first = pl.pallas_call(
    _mm_first_body,
    grid=(NPAD // _R,),
    in_specs=[_spec_rows, _spec_w, _spec_cnt],
    out_specs=_spec_rows,
    out_shape=_out_rows,
)

_mm_mid_leaky = pl.pallas_call(
    functools.partial(_mm_mid_body, leaky=True),
    grid=(NPAD // _R,),
    in_specs=[_spec_rows, _spec_rows, _spec_cnt, _spec_w, _spec_b],
    out_specs=_spec_rows,
    out_shape=_out_rows,
)

_mm_mid_plain = pl.pallas_call(
    functools.partial(_mm_mid_body, leaky=False),
    grid=(NPAD // _R,),
    in_specs=[_spec_rows, _spec_rows, _spec_cnt, _spec_w, _spec_b],
    out_specs=_spec_rows,
    out_shape=_out_rows,
)

_fin = pl.pallas_call(
    _fin_body,
    grid=(NPAD // _R,),
    in_specs=[_spec_rows, _spec_rows, _spec_cnt, _spec_b],
    out_specs=_spec_rows,
    out_shape=_out_rows,
)


def kernel(x, edge_index, W1, b1, W2, b2, W3, b3):
    ei = edge_index.astype(jnp.int32)
    pad = jnp.full((E_PAD - E,), N, jnp.int32)
    src_flat = jnp.concatenate([ei[0], pad])
    dst_flat = jnp.concatenate([ei[1], pad])
    srcg = src_flat.reshape(NS, ANCH, ACH)
    dstg = dst_flat.reshape(NS, ANCH, ACH)
    xp = jnp.pad(x, ((0, NPAD - N), (0, 0)))

    cnt = _deg_kernel(dst_flat.reshape(NW, NCH, CH))

    hs = _mm_first(xp, W1, cnt)
    agg = _agg_kernel(hs, srcg, dstg)
    hs = _mm_mid_leaky(agg, hs, cnt, W2, b1.reshape(1, D))
    agg = _agg_kernel(hs, srcg, dstg)
    hs = _mm_mid_plain(agg, hs, cnt, W3, b2.reshape(1, D))
    agg = _agg_kernel(hs, srcg, dstg)
    out = _fin(agg, hs, cnt, b3.reshape(1, D))
    return out[:N]


# speedup vs baseline: 1.2130x; 1.0170x over previous
import functools

import jax
import jax.numpy as jnp
from jax import lax
from jax.experimental import pallas as pl
from jax.experimental.pallas import tpu as pltpu
from jax.experimental.pallas import tpu_sc as plsc

N = 10000
D = 128
FH = 64
E = 320000
NC = 2
NS = 16
NW = NC * NS
NPAD = 10240
RPT = NPAD // NS

CH = 128
EPT_DEG = 10240
NCH = EPT_DEG // CH
E_PAD = NW * EPT_DEG

ACH = 64
EPT = E_PAD // NS
ANCH = EPT // ACH
QTR = ANCH // 4
NBUF = 2

_MESH = plsc.VectorSubcoreMesh(core_axis_name="c", subcore_axis_name="s")


def _deg_body(dstg_hbm, cnt_out, cnt_sh, idxd_v, ones_v, zer_v):
    c = lax.axis_index("c")
    s = lax.axis_index("s")
    w = c * NS + s
    for i in range(8):
        ones_v[pl.ds(i * 16, 16)] = jnp.ones((16,), jnp.float32)
    for i in range(RPT // 16):
        zer_v[pl.ds(i * 16, 16)] = jnp.zeros((16,), jnp.float32)
    pltpu.sync_copy(zer_v, cnt_sh.at[pl.ds(s * RPT, RPT)])
    pltpu.sync_copy(dstg_hbm.at[w], idxd_v)
    plsc.subcore_barrier()

    @pl.loop(0, NCH)
    def _chunk(g):
        pltpu.sync_copy(ones_v, cnt_sh.at[idxd_v.at[g]], add=True)

    plsc.subcore_barrier()
    pltpu.sync_copy(cnt_sh.at[pl.ds(s * RPT, RPT)],
                    cnt_out.at[c, pl.ds(s * RPT, RPT)])


_deg_kernel = pl.kernel(
    _deg_body,
    out_type=jax.ShapeDtypeStruct((NC, NPAD), jnp.float32),
    mesh=_MESH,
    scratch_types=[
        pltpu.VMEM_SHARED((NPAD,), jnp.float32),
        pltpu.VMEM((NCH, CH), jnp.int32),
        pltpu.VMEM((CH,), jnp.float32),
        pltpu.VMEM((RPT,), jnp.float32),
    ],
)


def _agg_body(hs_hbm, srcg_hbm, dstg_hbm, agg_out,
              hs_sh, agg_sh, idxs_v, idxd_v, rows0, rows1,
              sem0, sem1, ssem0, ssem1):
    c = lax.axis_index("c")
    s = lax.axis_index("s")
    rows = (rows0, rows1)
    sems = (sem0, sem1)
    ssems = (ssem0, ssem1)

    @pl.loop(0, ACH)
    def _zr(i):
        @pl.loop(0, FH // 16)
        def _zc(j):
            rows0[i, pl.ds(j * 16, 16)] = jnp.zeros((16,), jnp.float32)

    @pl.loop(0, RPT // ACH)
    def _z(j):
        pltpu.sync_copy(rows0, agg_sh.at[pl.ds(s * RPT + j * ACH, ACH)])
        pltpu.sync_copy(hs_hbm.at[pl.ds(s * RPT + j * ACH, ACH),
                                  pl.ds(c * FH, FH)],
                        hs_sh.at[pl.ds(s * RPT + j * ACH, ACH)])

    plsc.subcore_barrier()

    for h in range(ANCH // QTR):
        pltpu.sync_copy(srcg_hbm.at[s, pl.ds(h * QTR, QTR)], idxs_v)
        pltpu.sync_copy(dstg_hbm.at[s, pl.ds(h * QTR, QTR)], idxd_v)
        for b in range(NBUF):
            pltpu.async_copy(hs_sh.at[idxs_v.at[b]], rows[b], sems[b])

        @pl.loop(0, QTR, step=NBUF)
        def _chunks(g):
            for b in range(NBUF):
                pltpu.make_async_copy(hs_sh.at[idxs_v.at[0]],
                                      rows[b], sems[b]).wait()
                pltpu.async_copy(rows[b], agg_sh.at[idxd_v.at[g + b]],
                                 ssems[b], add=True)

                @pl.when(g + b + NBUF < QTR)
                def _(b=b, g=g):
                    pltpu.make_async_copy(rows[b], agg_sh.at[idxd_v.at[0]],
                                          ssems[b]).wait()
                    pltpu.async_copy(hs_sh.at[idxs_v.at[g + b + NBUF]],
                                     rows[b], sems[b])

        for b in range(NBUF):
            @pl.when(QTR - NBUF + b >= 0)
            def _(b=b):
                pltpu.make_async_copy(rows[b], agg_sh.at[idxd_v.at[0]],
                                      ssems[b]).wait()

    plsc.subcore_barrier()

    @pl.loop(0, RPT // ACH)
    def _wb(j):
        pltpu.sync_copy(agg_sh.at[pl.ds(s * RPT + j * ACH, ACH)], rows0)
        pltpu.sync_copy(rows0, agg_out.at[pl.ds(s * RPT + j * ACH, ACH),
                                          pl.ds(c * FH, FH)])


_agg_kernel = pl.kernel(
    _agg_body,
    out_type=jax.ShapeDtypeStruct((NPAD, D), jnp.float32),
    mesh=_MESH,
    compiler_params=pltpu.CompilerParams(use_tc_tiling_on_sc=False),
    scratch_types=[
        pltpu.VMEM_SHARED((NPAD, FH), jnp.float32),
        pltpu.VMEM_SHARED((NPAD, FH), jnp.float32),
        pltpu.VMEM((QTR, ACH), jnp.int32),
        pltpu.VMEM((QTR, ACH), jnp.int32),
        pltpu.VMEM((ACH, FH), jnp.float32),
        pltpu.VMEM((ACH, FH), jnp.float32),
        pltpu.SemaphoreType.DMA,
        pltpu.SemaphoreType.DMA,
        pltpu.SemaphoreType.DMA,
        pltpu.SemaphoreType.DMA,
    ],
)


_R = 2048


def _dinv_of(cnt_ref):
    return lax.rsqrt(cnt_ref[0, :] + cnt_ref[1, :] + 1.0)


def _mm_first_body(x_ref, w_ref, cnt_ref, out_ref):
    dinv = _dinv_of(cnt_ref)
    h = jnp.dot(x_ref[...], w_ref[...], preferred_element_type=jnp.float32)
    out_ref[...] = h * dinv[:, None]


def _mm_mid_body(agg_ref, hs_ref, cnt_ref, w_ref, b_ref, out_ref, *, leaky):
    dinv = _dinv_of(cnt_ref)
    a = agg_ref[...] + hs_ref[...]
    xn = dinv[:, None] * a + b_ref[...]
    if leaky:
        xn = jnp.where(xn >= 0, xn, 0.01 * xn)
    h = jnp.dot(xn, w_ref[...], preferred_element_type=jnp.float32)
    out_ref[...] = h * dinv[:, None]


def _fin_body(agg_ref, hs_ref, cnt_ref, b_ref, out_ref):
    dinv = _dinv_of(cnt_ref)
    a = agg_ref[...] + hs_ref[...]
    xn = dinv[:, None] * a + b_ref[...]
    out_ref[...] = jnp.where(xn >= 0, xn, 0.01 * xn)


_spec_rows = pl.BlockSpec((_R, D), lambda i: (i, 0))
_spec_w = pl.BlockSpec((D, D), lambda i: (0, 0))
_spec_cnt = pl.BlockSpec((NC, _R), lambda i: (0, i))
_spec_b = pl.BlockSpec((1, D), lambda i: (0, 0))
_out_rows = jax.ShapeDtypeStruct((NPAD, D), jnp.float32)

_mm_first = pl.pallas_call(
    _mm_first_body,
    grid=(NPAD // _R,),
    in_specs=[_spec_rows, _spec_w, _spec_cnt],
    out_specs=_spec_rows,
    out_shape=_out_rows,
)

_mm_mid_leaky = pl.pallas_call(
    functools.partial(_mm_mid_body, leaky=True),
    grid=(NPAD // _R,),
    in_specs=[_spec_rows, _spec_rows, _spec_cnt, _spec_w, _spec_b],
    out_specs=_spec_rows,
    out_shape=_out_rows,
)

_mm_mid_plain = pl.pallas_call(
    functools.partial(_mm_mid_body, leaky=False),
    grid=(NPAD // _R,),
    in_specs=[_spec_rows, _spec_rows, _spec_cnt, _spec_w, _spec_b],
    out_specs=_spec_rows,
    out_shape=_out_rows,
)

_fin = pl.pallas_call(
    _fin_body,
    grid=(NPAD // _R,),
    in_specs=[_spec_rows, _spec_rows, _spec_cnt, _spec_b],
    out_specs=_spec_rows,
    out_shape=_out_rows,
)


def kernel(x, edge_index, W1, b1, W2, b2, W3, b3):
    ei = edge_index.astype(jnp.int32)
    pad = jnp.full((E_PAD - E,), N, jnp.int32)
    src_flat = jnp.concatenate([ei[0], pad])
    dst_flat = jnp.concatenate([ei[1], pad])
    srcg = src_flat.reshape(NS, ANCH, ACH)
    dstg = dst_flat.reshape(NS, ANCH, ACH)
    xp = jnp.pad(x, ((0, NPAD - N), (0, 0)))

    cnt = _deg_kernel(dst_flat.reshape(NW, NCH, CH))

    hs = _mm_first(xp, W1, cnt)
    agg = _agg_kernel(hs, srcg, dstg)
    hs = _mm_mid_leaky(agg, hs, cnt, W2, b1.reshape(1, D))
    agg = _agg_kernel(hs, srcg, dstg)
    hs = _mm_mid_plain(agg, hs, cnt, W3, b2.reshape(1, D))
    agg = _agg_kernel(hs, srcg, dstg)
    out = _fin(agg, hs, cnt, b3.reshape(1, D))
    return out[:N]


# speedup vs baseline: 1.2330x; 1.0165x over previous
import functools

import jax
import jax.numpy as jnp
from jax import lax
from jax.experimental import pallas as pl
from jax.experimental.pallas import tpu as pltpu
from jax.experimental.pallas import tpu_sc as plsc

N = 10000
D = 128
FH = 64
E = 320000
NC = 2
NS = 16
NW = NC * NS
NPAD = 10240
RPT = NPAD // NS

CH = 128
EPT_DEG = 10240
NCH = EPT_DEG // CH
E_PAD = NW * EPT_DEG

ACH = 64
EPT = E_PAD // NS
ANCH = EPT // ACH
QTR = ANCH // 4
NBUF = 2

_MESH = plsc.VectorSubcoreMesh(core_axis_name="c", subcore_axis_name="s")


def _deg_body(dstg_hbm, cnt_out, cnt_sh, idxd_v, ones_v, zer_v):
    c = lax.axis_index("c")
    s = lax.axis_index("s")
    w = c * NS + s
    for i in range(8):
        ones_v[pl.ds(i * 16, 16)] = jnp.ones((16,), jnp.float32)
    for i in range(RPT // 16):
        zer_v[pl.ds(i * 16, 16)] = jnp.zeros((16,), jnp.float32)
    pltpu.sync_copy(zer_v, cnt_sh.at[pl.ds(s * RPT, RPT)])
    pltpu.sync_copy(dstg_hbm.at[w], idxd_v)
    plsc.subcore_barrier()

    @pl.loop(0, NCH)
    def _chunk(g):
        pltpu.sync_copy(ones_v, cnt_sh.at[idxd_v.at[g]], add=True)

    plsc.subcore_barrier()
    pltpu.sync_copy(cnt_sh.at[pl.ds(s * RPT, RPT)],
                    cnt_out.at[c, pl.ds(s * RPT, RPT)])


_deg_kernel = pl.kernel(
    _deg_body,
    out_type=jax.ShapeDtypeStruct((NC, NPAD), jnp.float32),
    mesh=_MESH,
    scratch_types=[
        pltpu.VMEM_SHARED((NPAD,), jnp.float32),
        pltpu.VMEM((NCH, CH), jnp.int32),
        pltpu.VMEM((CH,), jnp.float32),
        pltpu.VMEM((RPT,), jnp.float32),
    ],
)


def _agg_body(hs_hbm, srcg_hbm, dstg_hbm, agg_out,
              hs_sh, agg_sh, idxs_v, idxd_v, rows0, rows1,
              sem0, sem1, ssem0, ssem1):
    c = lax.axis_index("c")
    s = lax.axis_index("s")
    rows = (rows0, rows1)
    sems = (sem0, sem1)
    ssems = (ssem0, ssem1)

    @pl.loop(0, ACH)
    def _zr(i):
        @pl.loop(0, FH // 16)
        def _zc(j):
            rows0[i, pl.ds(j * 16, 16)] = jnp.zeros((16,), jnp.float32)

    @pl.loop(0, RPT // ACH)
    def _z(j):
        pltpu.sync_copy(rows0, agg_sh.at[pl.ds(s * RPT + j * ACH, ACH)])
        pltpu.sync_copy(hs_hbm.at[pl.ds(s * RPT + j * ACH, ACH),
                                  pl.ds(c * FH, FH)],
                        hs_sh.at[pl.ds(s * RPT + j * ACH, ACH)])

    plsc.subcore_barrier()

    for h in range(ANCH // QTR):
        pltpu.sync_copy(srcg_hbm.at[s, pl.ds(h * QTR, QTR)], idxs_v)
        pltpu.sync_copy(dstg_hbm.at[s, pl.ds(h * QTR, QTR)], idxd_v)
        for b in range(NBUF):
            pltpu.async_copy(hs_sh.at[idxs_v.at[b]], rows[b], sems[b])

        @pl.loop(0, QTR, step=NBUF)
        def _chunks(g):
            for b in range(NBUF):
                pltpu.make_async_copy(hs_sh.at[idxs_v.at[0]],
                                      rows[b], sems[b]).wait()
                pltpu.async_copy(rows[b], agg_sh.at[idxd_v.at[g + b]],
                                 ssems[b], add=True)

                @pl.when(g + b + NBUF < QTR)
                def _(b=b, g=g):
                    pltpu.make_async_copy(rows[b], agg_sh.at[idxd_v.at[0]],
                                          ssems[b]).wait()
                    pltpu.async_copy(hs_sh.at[idxs_v.at[g + b + NBUF]],
                                     rows[b], sems[b])

        for b in range(NBUF):
            @pl.when(QTR - NBUF + b >= 0)
            def _(b=b):
                pltpu.make_async_copy(rows[b], agg_sh.at[idxd_v.at[0]],
                                      ssems[b]).wait()

    plsc.subcore_barrier()

    @pl.loop(0, RPT // ACH)
    def _wb(j):
        pltpu.sync_copy(agg_sh.at[pl.ds(s * RPT + j * ACH, ACH)], rows0)
        pltpu.sync_copy(rows0, agg_out.at[pl.ds(s * RPT + j * ACH, ACH),
                                          pl.ds(c * FH, FH)])


_agg_kernel = pl.kernel(
    _agg_body,
    out_type=jax.ShapeDtypeStruct((NPAD, D), jnp.float32),
    mesh=_MESH,
    compiler_params=pltpu.CompilerParams(use_tc_tiling_on_sc=False),
    scratch_types=[
        pltpu.VMEM_SHARED((NPAD, FH), jnp.float32),
        pltpu.VMEM_SHARED((NPAD, FH), jnp.float32),
        pltpu.VMEM((QTR, ACH), jnp.int32),
        pltpu.VMEM((QTR, ACH), jnp.int32),
        pltpu.VMEM((ACH, FH), jnp.float32),
        pltpu.VMEM((ACH, FH), jnp.float32),
        pltpu.SemaphoreType.DMA,
        pltpu.SemaphoreType.DMA,
        pltpu.SemaphoreType.DMA,
        pltpu.SemaphoreType.DMA,
    ],
)


_R = 5120


def _dinv_of(cnt_ref):
    return lax.rsqrt(cnt_ref[0, :] + cnt_ref[1, :] + 1.0)


def _mm_first_body(x_ref, w_ref, cnt_ref, out_ref):
    dinv = _dinv_of(cnt_ref)
    h = jnp.dot(x_ref[...], w_ref[...], preferred_element_type=jnp.float32)
    out_ref[...] = h * dinv[:, None]


def _mm_mid_body(agg_ref, hs_ref, cnt_ref, w_ref, b_ref, out_ref, *, leaky):
    dinv = _dinv_of(cnt_ref)
    a = agg_ref[...] + hs_ref[...]
    xn = dinv[:, None] * a + b_ref[...]
    if leaky:
        xn = jnp.where(xn >= 0, xn, 0.01 * xn)
    h = jnp.dot(xn, w_ref[...], preferred_element_type=jnp.float32)
    out_ref[...] = h * dinv[:, None]


def _fin_body(agg_ref, hs_ref, cnt_ref, b_ref, out_ref):
    dinv = _dinv_of(cnt_ref)
    a = agg_ref[...] + hs_ref[...]
    xn = dinv[:, None] * a + b_ref[...]
    out_ref[...] = jnp.where(xn >= 0, xn, 0.01 * xn)


_spec_rows = pl.BlockSpec((_R, D), lambda i: (i, 0))
_spec_w = pl.BlockSpec((D, D), lambda i: (0, 0))
_spec_cnt = pl.BlockSpec((NC, _R), lambda i: (0, i))
_spec_b = pl.BlockSpec((1, D), lambda i: (0, 0))
_out_rows = jax.ShapeDtypeStruct((NPAD, D), jnp.float32)

_mm_first = pl.pallas_call(
    _mm_first_body,
    grid=(NPAD // _R,),
    in_specs=[_spec_rows, _spec_w, _spec_cnt],
    out_specs=_spec_rows,
    out_shape=_out_rows,
)

_mm_mid_leaky = pl.pallas_call(
    functools.partial(_mm_mid_body, leaky=True),
    grid=(NPAD // _R,),
    in_specs=[_spec_rows, _spec_rows, _spec_cnt, _spec_w, _spec_b],
    out_specs=_spec_rows,
    out_shape=_out_rows,
)

_mm_mid_plain = pl.pallas_call(
    functools.partial(_mm_mid_body, leaky=False),
    grid=(NPAD // _R,),
    in_specs=[_spec_rows, _spec_rows, _spec_cnt, _spec_w, _spec_b],
    out_specs=_spec_rows,
    out_shape=_out_rows,
)

_fin = pl.pallas_call(
    _fin_body,
    grid=(NPAD // _R,),
    in_specs=[_spec_rows, _spec_rows, _spec_cnt, _spec_b],
    out_specs=_spec_rows,
    out_shape=_out_rows,
)


def kernel(x, edge_index, W1, b1, W2, b2, W3, b3):
    ei = edge_index.astype(jnp.int32)
    pad = jnp.full((E_PAD - E,), N, jnp.int32)
    src_flat = jnp.concatenate([ei[0], pad])
    dst_flat = jnp.concatenate([ei[1], pad])
    srcg = src_flat.reshape(NS, ANCH, ACH)
    dstg = dst_flat.reshape(NS, ANCH, ACH)
    xp = jnp.pad(x, ((0, NPAD - N), (0, 0)))

    cnt = _deg_kernel(dst_flat.reshape(NW, NCH, CH))

    hs = _mm_first(xp, W1, cnt)
    agg = _agg_kernel(hs, srcg, dstg)
    hs = _mm_mid_leaky(agg, hs, cnt, W2, b1.reshape(1, D))
    agg = _agg_kernel(hs, srcg, dstg)
    hs = _mm_mid_plain(agg, hs, cnt, W3, b2.reshape(1, D))
    agg = _agg_kernel(hs, srcg, dstg)
    out = _fin(agg, hs, cnt, b3.reshape(1, D))
    return out[:N]
